# baseline (speedup 1.0000x reference)
import functools
import jax
import jax.numpy as jnp
from jax import lax
from jax.experimental import pallas as pl
from jax.experimental.pallas import tpu as pltpu
from jax.experimental.pallas import tpu_sc as plsc

_NC = 2
_NS = 16
_NW = _NC * _NS
_LANES = 16


def _sc_pe_add(x, pos_embed):
    B, S, D = x.shape
    SEQ_PER_W = S // _NW
    C = 32
    CHUNKS = SEQ_PER_W // C
    CW = C * D
    x1 = x.reshape(B * S * D)
    pe1 = pos_embed.reshape(S * D)

    mesh = plsc.VectorSubcoreMesh(core_axis_name="c", subcore_axis_name="s")

    @functools.partial(
        pl.kernel,
        out_type=jax.ShapeDtypeStruct((B * S * D,), jnp.float32),
        mesh=mesh,
        scratch_types=[
            pltpu.VMEM((CW,), jnp.float32),
            pltpu.VMEM((CW,), jnp.float32),
        ],
    )
    def sc_add(x_hbm, pe_hbm, out_hbm, pe_v, x_v):
        wid = lax.axis_index("s") * _NC + lax.axis_index("c")
        seq0 = wid * SEQ_PER_W

        def chunk_body(ci, carry):
            sbase = (seq0 + ci * C) * D
            pltpu.sync_copy(pe_hbm.at[pl.ds(sbase, CW)], pe_v)

            def batch_body(b, carry2):
                rbase = b * (S * D) + sbase
                pltpu.sync_copy(x_hbm.at[pl.ds(rbase, CW)], x_v)

                def vec_body(i, carry3):
                    sl = pl.ds(i * _LANES, _LANES)
                    x_v[sl] = x_v[sl] + pe_v[sl]
                    return carry3

                lax.fori_loop(0, CW // _LANES, vec_body, 0, unroll=8)
                pltpu.sync_copy(x_v, out_hbm.at[pl.ds(rbase, CW)])
                return carry2

            return lax.fori_loop(0, B, batch_body, carry)

        lax.fori_loop(0, CHUNKS, chunk_body, 0)

    return sc_add(x1, pe1).reshape(B, S, D)


def kernel(x, pos_embed):
    return _sc_pe_add(x, pos_embed)


# speedup vs baseline: 1.1040x; 1.1040x over previous
import functools
import jax
import jax.numpy as jnp
from jax import lax
from jax.experimental import pallas as pl
from jax.experimental.pallas import tpu as pltpu
from jax.experimental.pallas import tpu_sc as plsc

_NC = 2
_NS = 16
_NW = _NC * _NS
_LANES = 16


def _sc_pe_add(x, pos_embed):
    B, S, D = x.shape
    SEQ_PER_W = S // _NW
    C = 16
    CHUNKS = SEQ_PER_W // C
    CW = C * D
    NSTEPS = CHUNKS * B
    x1 = x.reshape(B * S * D)
    pe1 = pos_embed.reshape(S * D)

    mesh = plsc.VectorSubcoreMesh(core_axis_name="c", subcore_axis_name="s")

    @functools.partial(
        pl.kernel,
        out_type=jax.ShapeDtypeStruct((B * S * D,), jnp.float32),
        mesh=mesh,
        scratch_types=[
            pltpu.VMEM((CW,), jnp.float32),
            pltpu.VMEM((CW,), jnp.float32),
            pltpu.VMEM((CW,), jnp.float32),
            pltpu.VMEM((CW,), jnp.float32),
            pltpu.SemaphoreType.DMA,
            pltpu.SemaphoreType.DMA,
            pltpu.SemaphoreType.DMA,
            pltpu.SemaphoreType.DMA,
            pltpu.SemaphoreType.DMA,
            pltpu.SemaphoreType.DMA,
        ],
    )
    def sc_add(x_hbm, pe_hbm, out_hbm, pe0, pe1v, xb0, xb1,
               spe0, spe1, sld0, sld1, sst0, sst1):
        wid = lax.axis_index("s") * _NC + lax.axis_index("c")
        seq0 = wid * SEQ_PER_W

        pe_bufs = (pe0, pe1v)
        pe_sems = (spe0, spe1)
        x_bufs = (xb0, xb1)
        ld_sems = (sld0, sld1)
        st_sems = (sst0, sst1)

        def pe_base(ci):
            return (seq0 + ci * C) * D

        def x_base(step):
            ci, b = divmod(step, B)
            return b * (S * D) + pe_base(ci)

        def start_pe(ci):
            buf = pe_bufs[ci % 2]
            cp = pltpu.make_async_copy(
                pe_hbm.at[pl.ds(pe_base(ci), CW)], buf, pe_sems[ci % 2])
            cp.start()
            return cp

        def start_x(step):
            buf = x_bufs[step % 2]
            cp = pltpu.make_async_copy(
                x_hbm.at[pl.ds(x_base(step), CW)], buf, ld_sems[step % 2])
            cp.start()
            return cp

        def start_store(step):
            buf = x_bufs[step % 2]
            cp = pltpu.make_async_copy(
                buf, out_hbm.at[pl.ds(x_base(step), CW)], st_sems[step % 2])
            cp.start()
            return cp

        pe_cp = [None, None]
        pe_cp[0] = start_pe(0)
        x_cp = [None, None]
        x_cp[0] = start_x(0)
        st_cp = [None, None]

        for step in range(NSTEPS):
            ci, b = divmod(step, B)
            par = step % 2

            if b == 0 and ci + 1 < CHUNKS:
                pe_cp[(ci + 1) % 2] = start_pe(ci + 1)

            if step + 1 < NSTEPS:
                if st_cp[(step + 1) % 2] is not None:
                    st_cp[(step + 1) % 2].wait()
                    st_cp[(step + 1) % 2] = None
                x_cp[(step + 1) % 2] = start_x(step + 1)

            if b == 0:
                pe_cp[ci % 2].wait()
            x_cp[par].wait()

            xb = x_bufs[par]
            peb = pe_bufs[ci % 2]

            def vec_body(i, carry, xb=xb, peb=peb):
                sl = pl.ds(i * _LANES, _LANES)
                xb[sl] = xb[sl] + peb[sl]
                return carry

            lax.fori_loop(0, CW // _LANES, vec_body, 0, unroll=8)
            st_cp[par] = start_store(step)

        for p in range(2):
            if st_cp[p] is not None:
                st_cp[p].wait()

    return sc_add(x1, pe1).reshape(B, S, D)


def kernel(x, pos_embed):
    return _sc_pe_add(x, pos_embed)


# speedup vs baseline: 1.6673x; 1.5103x over previous
import functools
import jax
import jax.numpy as jnp
from jax import lax
from jax.experimental import pallas as pl
from jax.experimental.pallas import tpu as pltpu
from jax.experimental.pallas import tpu_sc as plsc

_NC = 2
_NS = 16
_NW = _NC * _NS
_LANES = 16


def _sc_pe_add(x, pos_embed):
    B, S, D = x.shape
    SEQ_PER_W = S // _NW
    C = 16
    CHUNKS = SEQ_PER_W // C
    CW = C * D
    NSTEPS = CHUNKS * B
    x1 = x.reshape(B * S * D)
    pe1 = pos_embed.reshape(S * D)

    mesh = plsc.VectorSubcoreMesh(core_axis_name="c", subcore_axis_name="s")

    @functools.partial(
        pl.kernel,
        out_type=jax.ShapeDtypeStruct((B * S * D,), jnp.float32),
        mesh=mesh,
        scratch_types=[
            pltpu.VMEM((CW,), jnp.float32),
            pltpu.VMEM((CW,), jnp.float32),
            pltpu.VMEM((CW,), jnp.float32),
            pltpu.VMEM((CW,), jnp.float32),
            pltpu.SemaphoreType.DMA,
            pltpu.SemaphoreType.DMA,
            pltpu.SemaphoreType.DMA,
            pltpu.SemaphoreType.DMA,
            pltpu.SemaphoreType.DMA,
            pltpu.SemaphoreType.DMA,
        ],
    )
    def sc_add(x_hbm, pe_hbm, out_hbm, pe0, pe1v, xb0, xb1,
               spe0, spe1, sld0, sld1, sst0, sst1):
        wid = lax.axis_index("s") * _NC + lax.axis_index("c")
        seq0 = wid * SEQ_PER_W

        pe_bufs = (pe0, pe1v)
        pe_sems = (spe0, spe1)
        x_bufs = (xb0, xb1)
        ld_sems = (sld0, sld1)
        st_sems = (sst0, sst1)

        def pe_base(ci):
            return (seq0 + ci * C) * D

        def x_base(step):
            ci, b = divmod(step, B)
            return b * (S * D) + pe_base(ci)

        def start_pe(ci):
            buf = pe_bufs[ci % 2]
            cp = pltpu.make_async_copy(
                pe_hbm.at[pl.ds(pe_base(ci), CW)], buf, pe_sems[ci % 2])
            cp.start()
            return cp

        def start_x(step):
            buf = x_bufs[step % 2]
            cp = pltpu.make_async_copy(
                x_hbm.at[pl.ds(x_base(step), CW)], buf, ld_sems[step % 2])
            cp.start()
            return cp

        def start_store(step):
            buf = x_bufs[step % 2]
            cp = pltpu.make_async_copy(
                buf, out_hbm.at[pl.ds(x_base(step), CW)], st_sems[step % 2])
            cp.start()
            return cp

        pe_cp = [None, None]
        pe_cp[0] = start_pe(0)
        x_cp = [None, None]
        x_cp[0] = start_x(0)
        st_cp = [None, None]

        for step in range(NSTEPS):
            ci, b = divmod(step, B)
            par = step % 2

            if b == 0 and ci + 1 < CHUNKS:
                pe_cp[(ci + 1) % 2] = start_pe(ci + 1)

            if step + 1 < NSTEPS:
                if st_cp[(step + 1) % 2] is not None:
                    st_cp[(step + 1) % 2].wait()
                    st_cp[(step + 1) % 2] = None
                x_cp[(step + 1) % 2] = start_x(step + 1)

            if b == 0:
                pe_cp[ci % 2].wait()
            x_cp[par].wait()

            xb = x_bufs[par]
            peb = pe_bufs[ci % 2]

            @plsc.parallel_loop(0, CW, step=_LANES, unroll=8)
            def vec_body(i, xb=xb, peb=peb):
                sl = pl.ds(i, _LANES)
                xb[sl] = xb[sl] + peb[sl]
            st_cp[par] = start_store(step)

        for p in range(2):
            if st_cp[p] is not None:
                st_cp[p].wait()

    return sc_add(x1, pe1).reshape(B, S, D)


def kernel(x, pos_embed):
    return _sc_pe_add(x, pos_embed)


# speedup vs baseline: 3.7212x; 2.2318x over previous
import functools
import jax
import jax.numpy as jnp
from jax import lax
from jax.experimental import pallas as pl
from jax.experimental.pallas import tpu as pltpu
from jax.experimental.pallas import tpu_sc as plsc

_NC = 2
_NS = 16
_NW = _NC * _NS
_LANES = 16


def _sc_pe_add(x, pos_embed):
    B, S, D = x.shape
    SEQ_PER_W = S // _NW
    C = 32
    CHUNKS = SEQ_PER_W // C
    NSTEPS = CHUNKS * B
    x2 = x.reshape(B * S, D)

    mesh = plsc.VectorSubcoreMesh(core_axis_name="c", subcore_axis_name="s")

    @functools.partial(
        pl.kernel,
        out_type=jax.ShapeDtypeStruct((B * S, D), jnp.float32),
        mesh=mesh,
        scratch_types=[
            pltpu.VMEM((C, D), jnp.float32),
            pltpu.VMEM((C, D), jnp.float32),
            pltpu.VMEM((C, D), jnp.float32),
            pltpu.SemaphoreType.DMA,
            pltpu.SemaphoreType.DMA,
            pltpu.SemaphoreType.DMA,
            pltpu.SemaphoreType.DMA,
            pltpu.SemaphoreType.DMA,
        ],
    )
    def sc_add(x_hbm, pe_hbm, out_hbm, peb, xb0, xb1,
               spe, sld0, sld1, sst0, sst1):
        wid = lax.axis_index("s") * _NC + lax.axis_index("c")
        seq0 = wid * SEQ_PER_W

        x_bufs = (xb0, xb1)
        ld_sems = (sld0, sld1)
        st_sems = (sst0, sst1)

        def x_row(step):
            ci, b = divmod(step, B)
            return b * S + seq0 + ci * C

        def start_pe(ci):
            cp = pltpu.make_async_copy(
                pe_hbm.at[pl.ds(seq0 + ci * C, C)], peb, spe)
            cp.start()
            return cp

        def start_x(step):
            cp = pltpu.make_async_copy(
                x_hbm.at[pl.ds(x_row(step), C)], x_bufs[step % 2],
                ld_sems[step % 2])
            cp.start()
            return cp

        def start_store(step):
            cp = pltpu.make_async_copy(
                x_bufs[step % 2], out_hbm.at[pl.ds(x_row(step), C)],
                st_sems[step % 2])
            cp.start()
            return cp

        pe_cp = start_pe(0)
        x_cp = [start_x(0), None]
        st_cp = [None, None]

        for step in range(NSTEPS):
            ci, b = divmod(step, B)
            par = step % 2

            if step + 1 < NSTEPS:
                if st_cp[(step + 1) % 2] is not None:
                    st_cp[(step + 1) % 2].wait()
                    st_cp[(step + 1) % 2] = None
                x_cp[(step + 1) % 2] = start_x(step + 1)

            if b == 0:
                pe_cp.wait()
            x_cp[par].wait()

            xb = x_bufs[par]

            @plsc.parallel_loop(0, C, step=1)
            def row_body(r, xb=xb):
                for c in range(0, D, _LANES):
                    sl = pl.ds(c, _LANES)
                    xb[r, sl] = xb[r, sl] + peb[r, sl]

            if b == B - 1 and ci + 1 < CHUNKS:
                pe_cp = start_pe(ci + 1)

            st_cp[par] = start_store(step)

        for p in range(2):
            if st_cp[p] is not None:
                st_cp[p].wait()

    return sc_add(x2, pos_embed).reshape(B, S, D)


def kernel(x, pos_embed):
    return _sc_pe_add(x, pos_embed)


# speedup vs baseline: 7.7773x; 2.0900x over previous
import jax
import jax.numpy as jnp
from jax.experimental import pallas as pl


def _pe_add_kernel(xa_ref, xb_ref, pe_ref, o_ref):
    o_ref[0] = xa_ref[0] + pe_ref[...]
    o_ref[1] = xb_ref[0] + pe_ref[...]


def kernel(x, pos_embed):
    B, S, D = x.shape
    BS = 1024
    return pl.pallas_call(
        _pe_add_kernel,
        grid=(S // BS, B // 2),
        in_specs=[
            pl.BlockSpec((1, BS, D), lambda s, b: (2 * b, s, 0)),
            pl.BlockSpec((1, BS, D), lambda s, b: (2 * b + 1, s, 0)),
            pl.BlockSpec((BS, D), lambda s, b: (s, 0)),
        ],
        out_specs=pl.BlockSpec((2, BS, D), lambda s, b: (b, s, 0)),
        out_shape=jax.ShapeDtypeStruct(x.shape, x.dtype),
    )(x, x, pos_embed)
